# baseline (speedup 1.0000x reference)
import functools

import jax
import jax.numpy as jnp
from jax import lax
from jax.experimental import pallas as pl
from jax.experimental.pallas import tpu as pltpu
from jax.experimental.pallas import tpu_sc as plsc

N = 100000
Z = 100000
S = 4
U = 32
D = 3
F = S * U
B = 160
NBLK = N // B
NW = 32
PAIRS = 10
L = 16
H = U // L


def _compute_block(g_ref, x_ref, o_ref, cb_v):
    for o in range(S):
        cb = [[cb_v[d, o, s, :] for s in range(S)] for d in range(D)]

        def row(r, _):
            g = [g_ref[r, pl.ds(j * L, L)] for j in range(S * H)]
            for h in range(H):
                j = o * H + h
                xo = x_ref[r, pl.ds(j * L, L)]
                m = [None] * D
                for d in range(D):
                    acc = cb[d][0] * g[0 * H + h]
                    for s in range(1, S):
                        acc = acc + cb[d][s] * g[s * H + h]
                    m[d] = acc
                r2 = m[D - 1]
                for d in range(D - 2, -1, -1):
                    r2 = r2 * xo + m[d]
                o_ref[r, pl.ds(j * L, L)] = r2 * xo
            return _

        lax.fori_loop(0, B, row, None)


def _body(x0_hbm, i0_hbm, x1_hbm, cb_hbm, out_hbm,
          idx0, idx1, g0, g1, xx0, xx1, oo0, oo1, cb_v,
          si0, si1, sg0, sg1, sx0, sx1, so0, so1):
    wid = lax.axis_index("s") * 2 + lax.axis_index("c")
    idx = (idx0, idx1)
    gg = (g0, g1)
    xx = (xx0, xx1)
    oo = (oo0, oo1)
    si = (si0, si1)
    sg = (sg0, sg1)
    sx = (sx0, sx1)
    so = (so0, so1)

    pltpu.sync_copy(cb_hbm, cb_v)

    def fire_idx(t, p):
        blk = wid + t * NW

        @pl.when(blk < NBLK)
        def _():
            pltpu.async_copy(i0_hbm.at[pl.ds(blk * B, B)], idx[p], si[p])

    def wait_idx(t, p):
        blk = wid + t * NW

        @pl.when(blk < NBLK)
        def _():
            pltpu.make_async_copy(i0_hbm.at[pl.ds(blk * B, B)], idx[p],
                                  si[p]).wait()

    def fire_in(t, b):
        blk = wid + t * NW

        @pl.when(blk < NBLK)
        def _():
            pltpu.async_copy(x0_hbm.at[idx[b]], gg[b], sg[b])
            pltpu.async_copy(x1_hbm.at[pl.ds(blk * B, B)], xx[b], sx[b])

    def wait_in(t, b):
        blk = wid + t * NW

        @pl.when(blk < NBLK)
        def _():
            pltpu.make_async_copy(x0_hbm.at[idx[b]], gg[b], sg[b]).wait()
            pltpu.make_async_copy(x1_hbm.at[pl.ds(blk * B, B)], xx[b],
                                  sx[b]).wait()

    def fire_out(t, b):
        blk = wid + t * NW

        @pl.when(blk < NBLK)
        def _():
            pltpu.async_copy(oo[b], out_hbm.at[pl.ds(blk * B, B)], so[b])

    def wait_out(t, b):
        blk = wid + t * NW

        @pl.when((t >= 0) & (blk < NBLK))
        def _():
            pltpu.make_async_copy(oo[b], out_hbm.at[pl.ds(blk * B, B)],
                                  so[b]).wait()

    def compute(t, b):
        blk = wid + t * NW

        @pl.when(blk < NBLK)
        def _():
            _compute_block(gg[b], xx[b], oo[b], cb_v)

    fire_idx(0, 0)
    fire_idx(1, 1)
    wait_idx(0, 0)
    fire_in(0, 0)

    def pair(i, _):
        for b in range(2):
            t = 2 * i + b
            wait_in(t, b)
            wait_idx(t + 1, 1 - b)
            fire_in(t + 1, 1 - b)
            fire_idx(t + 2, b)
            wait_out(t - 2, b)
            compute(t, b)
            fire_out(t, b)
        return _

    lax.fori_loop(0, PAIRS, pair, None)
    wait_out(2 * PAIRS - 2, 0)
    wait_out(2 * PAIRS - 1, 1)


@jax.jit
def _run(x0, i0, x1, cb):
    mesh = plsc.VectorSubcoreMesh(core_axis_name="c", subcore_axis_name="s")
    fn = functools.partial(
        pl.kernel,
        mesh=mesh,
        out_type=jax.ShapeDtypeStruct((N, F), jnp.float32),
        scratch_types=[
            pltpu.VMEM((B,), jnp.int32),
            pltpu.VMEM((B,), jnp.int32),
            pltpu.VMEM((B, F), jnp.float32),
            pltpu.VMEM((B, F), jnp.float32),
            pltpu.VMEM((B, F), jnp.float32),
            pltpu.VMEM((B, F), jnp.float32),
            pltpu.VMEM((B, F), jnp.float32),
            pltpu.VMEM((B, F), jnp.float32),
            pltpu.VMEM((D, S, S, L), jnp.float32),
            pltpu.SemaphoreType.DMA,
            pltpu.SemaphoreType.DMA,
            pltpu.SemaphoreType.DMA,
            pltpu.SemaphoreType.DMA,
            pltpu.SemaphoreType.DMA,
            pltpu.SemaphoreType.DMA,
            pltpu.SemaphoreType.DMA,
            pltpu.SemaphoreType.DMA,
        ],
    )(_body)
    return fn(x0, i0, x1, cb)


def kernel(x0, i0, x1, C):
    i0 = i0.astype(jnp.int32)
    cb = jnp.broadcast_to(C[:, :, :, None], (D, S, S, L)).astype(jnp.float32)
    return _run(x0, i0, x1, cb)


# speedup vs baseline: 1.0216x; 1.0216x over previous
import functools

import jax
import jax.numpy as jnp
from jax import lax
from jax.experimental import pallas as pl
from jax.experimental.pallas import tpu as pltpu
from jax.experimental.pallas import tpu_sc as plsc

N = 100000
Z = 100000
S = 4
U = 32
D = 3
F = S * U
B = 160
NBLK = N // B
NW = 32
PAIRS = 10
L = 16
H = U // L


def _compute_block(g_ref, x_ref, o_ref, cb_v):
    for o in range(S):
        cb = [[cb_v[d, o, s, :] for s in range(S)] for d in range(D)]

        def row(i, _):
            for r in (2 * i, 2 * i + 1):
                g = [g_ref[r, pl.ds(j * L, L)] for j in range(S * H)]
                for h in range(H):
                    j = o * H + h
                    xo = x_ref[r, pl.ds(j * L, L)]
                    m = [None] * D
                    for d in range(D):
                        acc = cb[d][0] * g[0 * H + h]
                        for s in range(1, S):
                            acc = acc + cb[d][s] * g[s * H + h]
                        m[d] = acc
                    r2 = m[D - 1]
                    for d in range(D - 2, -1, -1):
                        r2 = r2 * xo + m[d]
                    o_ref[r, pl.ds(j * L, L)] = r2 * xo
            return _

        lax.fori_loop(0, B // 2, row, None)


def _body(x0_hbm, i0_hbm, x1_hbm, cb_hbm, out_hbm,
          idx0, idx1, g0, g1, xx0, xx1, oo0, oo1, cb_v,
          si0, si1, sg0, sg1, sx0, sx1, so0, so1):
    wid = lax.axis_index("s") * 2 + lax.axis_index("c")
    idx = (idx0, idx1)
    gg = (g0, g1)
    xx = (xx0, xx1)
    oo = (oo0, oo1)
    si = (si0, si1)
    sg = (sg0, sg1)
    sx = (sx0, sx1)
    so = (so0, so1)

    pltpu.sync_copy(cb_hbm, cb_v)

    def fire_idx(t, p):
        blk = wid + t * NW

        @pl.when(blk < NBLK)
        def _():
            pltpu.async_copy(i0_hbm.at[pl.ds(blk * B, B)], idx[p], si[p])

    def wait_idx(t, p):
        blk = wid + t * NW

        @pl.when(blk < NBLK)
        def _():
            pltpu.make_async_copy(i0_hbm.at[pl.ds(blk * B, B)], idx[p],
                                  si[p]).wait()

    def fire_in(t, b):
        blk = wid + t * NW

        @pl.when(blk < NBLK)
        def _():
            pltpu.async_copy(x0_hbm.at[idx[b]], gg[b], sg[b])
            pltpu.async_copy(x1_hbm.at[pl.ds(blk * B, B)], xx[b], sx[b])

    def wait_in(t, b):
        blk = wid + t * NW

        @pl.when(blk < NBLK)
        def _():
            pltpu.make_async_copy(x0_hbm.at[idx[b]], gg[b], sg[b]).wait()
            pltpu.make_async_copy(x1_hbm.at[pl.ds(blk * B, B)], xx[b],
                                  sx[b]).wait()

    def fire_out(t, b):
        blk = wid + t * NW

        @pl.when(blk < NBLK)
        def _():
            pltpu.async_copy(oo[b], out_hbm.at[pl.ds(blk * B, B)], so[b])

    def wait_out(t, b):
        blk = wid + t * NW

        @pl.when((t >= 0) & (blk < NBLK))
        def _():
            pltpu.make_async_copy(oo[b], out_hbm.at[pl.ds(blk * B, B)],
                                  so[b]).wait()

    def compute(t, b):
        blk = wid + t * NW

        @pl.when(blk < NBLK)
        def _():
            _compute_block(gg[b], xx[b], oo[b], cb_v)

    fire_idx(0, 0)
    fire_idx(1, 1)
    wait_idx(0, 0)
    fire_in(0, 0)

    def pair(i, _):
        for b in range(2):
            t = 2 * i + b
            wait_in(t, b)
            wait_idx(t + 1, 1 - b)
            fire_in(t + 1, 1 - b)
            fire_idx(t + 2, b)
            wait_out(t - 2, b)
            compute(t, b)
            fire_out(t, b)
        return _

    lax.fori_loop(0, PAIRS, pair, None)
    wait_out(2 * PAIRS - 2, 0)
    wait_out(2 * PAIRS - 1, 1)


@jax.jit
def _run(x0, i0, x1, cb):
    mesh = plsc.VectorSubcoreMesh(core_axis_name="c", subcore_axis_name="s")
    fn = functools.partial(
        pl.kernel,
        mesh=mesh,
        out_type=jax.ShapeDtypeStruct((N, F), jnp.float32),
        scratch_types=[
            pltpu.VMEM((B,), jnp.int32),
            pltpu.VMEM((B,), jnp.int32),
            pltpu.VMEM((B, F), jnp.float32),
            pltpu.VMEM((B, F), jnp.float32),
            pltpu.VMEM((B, F), jnp.float32),
            pltpu.VMEM((B, F), jnp.float32),
            pltpu.VMEM((B, F), jnp.float32),
            pltpu.VMEM((B, F), jnp.float32),
            pltpu.VMEM((D, S, S, L), jnp.float32),
            pltpu.SemaphoreType.DMA,
            pltpu.SemaphoreType.DMA,
            pltpu.SemaphoreType.DMA,
            pltpu.SemaphoreType.DMA,
            pltpu.SemaphoreType.DMA,
            pltpu.SemaphoreType.DMA,
            pltpu.SemaphoreType.DMA,
            pltpu.SemaphoreType.DMA,
        ],
    )(_body)
    return fn(x0, i0, x1, cb)


def kernel(x0, i0, x1, C):
    i0 = i0.astype(jnp.int32)
    cb = jnp.broadcast_to(C[:, :, :, None], (D, S, S, L)).astype(jnp.float32)
    return _run(x0, i0, x1, cb)
